# Initial kernel scaffold; baseline (speedup 1.0000x reference)
import jax
import jax.numpy as jnp
from jax.experimental import pallas as pl


def kernel(x, edge_index, edge_attr, global_attr, W1, b1, W2, b2):
    raise NotImplementedError("write your pallas kernel here")


# speedup vs baseline: 3.7393x; 3.7393x over previous
import functools

import jax
import jax.numpy as jnp
from jax import lax
from jax.experimental import pallas as pl
from jax.experimental.pallas import tpu as pltpu
from jax.experimental.pallas import tpu_sc as plsc

N_NODES = 10000
N_EDGES = 320000
D_EDGE = 16
D_NODE = 128
LATENT = 32
D_OUT = 128

NUM_CORES = 2
NUM_SUBCORES = 16
NUM_TILES = NUM_CORES * NUM_SUBCORES

EDGES_PER_TILE = N_EDGES // NUM_TILES
CHUNK = 128
NFULL = EDGES_PER_TILE // CHUNK
REM = EDGES_PER_TILE - NFULL * CHUNK
STRIPE = 624
TAIL = N_NODES - NUM_SUBCORES * STRIPE


def _make_sc_segment_sum():
  mesh = plsc.VectorSubcoreMesh(core_axis_name="c", subcore_axis_name="s")

  @functools.partial(
      pl.kernel,
      mesh=mesh,
      compiler_params=pltpu.CompilerParams(use_tc_tiling_on_sc=False),
      out_type=jax.ShapeDtypeStruct((NUM_CORES, N_NODES, D_EDGE), jnp.float32),
      scratch_types=[
          pltpu.VMEM((CHUNK,), jnp.int32),
          pltpu.VMEM((CHUNK, D_EDGE), jnp.float32),
          pltpu.VMEM((REM,), jnp.int32),
          pltpu.VMEM((REM, D_EDGE), jnp.float32),
          pltpu.VMEM((STRIPE, D_EDGE), jnp.float32),
          pltpu.VMEM_SHARED((N_NODES, D_EDGE), jnp.float32),
      ],
  )
  def seg(recv_hbm, attr_hbm, out_hbm, idx_v, rows_v, idxr_v, rowsr_v, zbuf,
          acc_sh):
    c = lax.axis_index("c")
    s = lax.axis_index("s")
    wid = s * NUM_CORES + c
    base = wid * EDGES_PER_TILE
    stripe = s * STRIPE

    zeros16 = jnp.zeros((D_EDGE,), jnp.float32)

    def zrow(i, carry):
      zbuf[i, :] = zeros16
      return carry

    lax.fori_loop(0, STRIPE, zrow, 0)
    pltpu.sync_copy(zbuf, acc_sh.at[pl.ds(stripe, STRIPE)])

    @pl.when(s == NUM_SUBCORES - 1)
    def _zero_tail():
      pltpu.sync_copy(
          zbuf.at[pl.ds(0, TAIL)], acc_sh.at[pl.ds(N_NODES - TAIL, TAIL)])

    plsc.subcore_barrier()

    def body(j, carry):
      off = base + j * CHUNK
      pltpu.sync_copy(recv_hbm.at[pl.ds(off, CHUNK)], idx_v)
      pltpu.sync_copy(attr_hbm.at[pl.ds(off, CHUNK)], rows_v)
      pltpu.sync_copy(rows_v, acc_sh.at[idx_v], add=True)
      return carry

    lax.fori_loop(0, NFULL, body, 0)

    offr = base + NFULL * CHUNK
    pltpu.sync_copy(recv_hbm.at[pl.ds(offr, REM)], idxr_v)
    pltpu.sync_copy(attr_hbm.at[pl.ds(offr, REM)], rowsr_v)
    pltpu.sync_copy(rowsr_v, acc_sh.at[idxr_v], add=True)

    plsc.subcore_barrier()
    pltpu.sync_copy(
        acc_sh.at[pl.ds(stripe, STRIPE)],
        out_hbm.at[c, pl.ds(stripe, STRIPE)],
    )

    @pl.when(s == NUM_SUBCORES - 1)
    def _write_tail():
      pltpu.sync_copy(
          acc_sh.at[pl.ds(N_NODES - TAIL, TAIL)],
          out_hbm.at[c, pl.ds(N_NODES - TAIL, TAIL)],
      )

  return seg


_sc_segment_sum = _make_sc_segment_sum()

ROW_BLOCK = 1000
N_BLOCKS = N_NODES // ROW_BLOCK


def _mlp_body(x_ref, p_ref, g_ref, w1_ref, b1_ref, w2_ref, b2_ref, o_ref):
  xw = jnp.dot(x_ref[...], w1_ref[0:D_NODE, :],
               preferred_element_type=jnp.float32)
  p = p_ref[0] + p_ref[1]
  pw = jnp.dot(p, w1_ref[D_NODE:D_NODE + D_EDGE, :],
               preferred_element_type=jnp.float32)
  gw = jnp.dot(g_ref[...], w1_ref[D_NODE + D_EDGE:, :],
               preferred_element_type=jnp.float32)
  h = jnp.maximum(xw + pw + gw + b1_ref[...], 0.0)
  o_ref[...] = jnp.dot(h, w2_ref[...],
                       preferred_element_type=jnp.float32) + b2_ref[...]


def _tc_mlp(x, parts, global_attr, W1, b1, W2, b2):
  return pl.pallas_call(
      _mlp_body,
      grid=(N_BLOCKS,),
      in_specs=[
          pl.BlockSpec((ROW_BLOCK, D_NODE), lambda i: (i, 0)),
          pl.BlockSpec((NUM_CORES, ROW_BLOCK, D_EDGE), lambda i: (0, i, 0)),
          pl.BlockSpec((1, D_NODE), lambda i: (0, 0)),
          pl.BlockSpec((D_NODE + D_EDGE + D_NODE, LATENT), lambda i: (0, 0)),
          pl.BlockSpec((1, LATENT), lambda i: (0, 0)),
          pl.BlockSpec((LATENT, D_OUT), lambda i: (0, 0)),
          pl.BlockSpec((1, D_OUT), lambda i: (0, 0)),
      ],
      out_specs=pl.BlockSpec((ROW_BLOCK, D_OUT), lambda i: (i, 0)),
      out_shape=jax.ShapeDtypeStruct((N_NODES, D_OUT), jnp.float32),
  )(x, parts, global_attr, W1, b1, W2, b2)


def kernel(x, edge_index, edge_attr, global_attr, W1, b1, W2, b2):
  recv = edge_index[1].astype(jnp.int32)
  parts = _sc_segment_sum(recv, edge_attr)
  return _tc_mlp(
      x,
      parts,
      global_attr,
      W1,
      b1.reshape(1, LATENT),
      W2,
      b2.reshape(1, D_OUT),
  )


# speedup vs baseline: 4.9486x; 1.3234x over previous
import functools

import jax
import jax.numpy as jnp
from jax import lax
from jax.experimental import pallas as pl
from jax.experimental.pallas import tpu as pltpu
from jax.experimental.pallas import tpu_sc as plsc

N_NODES = 10000
N_EDGES = 320000
D_EDGE = 16
D_NODE = 128
LATENT = 32
D_OUT = 128

NUM_CORES = 2
NUM_SUBCORES = 16
NUM_TILES = NUM_CORES * NUM_SUBCORES

EDGES_PER_TILE = N_EDGES // NUM_TILES
CHUNK = 128
NFULL = EDGES_PER_TILE // CHUNK
REM = EDGES_PER_TILE - NFULL * CHUNK
STRIPE = 624
TAIL = N_NODES - NUM_SUBCORES * STRIPE


def _make_sc_segment_sum():
  mesh = plsc.VectorSubcoreMesh(core_axis_name="c", subcore_axis_name="s")

  @functools.partial(
      pl.kernel,
      mesh=mesh,
      compiler_params=pltpu.CompilerParams(use_tc_tiling_on_sc=False),
      out_type=jax.ShapeDtypeStruct((NUM_CORES, N_NODES, D_EDGE), jnp.float32),
      scratch_types=[
          pltpu.VMEM((EDGES_PER_TILE,), jnp.int32),
          pltpu.VMEM((CHUNK, D_EDGE), jnp.float32),
          pltpu.VMEM((CHUNK, D_EDGE), jnp.float32),
          pltpu.VMEM((REM, D_EDGE), jnp.float32),
          pltpu.VMEM((STRIPE, D_EDGE), jnp.float32),
          pltpu.VMEM_SHARED((N_NODES, D_EDGE), jnp.float32),
          pltpu.SemaphoreType.DMA,
          pltpu.SemaphoreType.DMA,
      ],
  )
  def seg(recv_hbm, attr_hbm, out_hbm, idx_all, rows0, rows1, rowsr_v, zbuf,
          acc_sh, sem0, sem1):
    c = lax.axis_index("c")
    s = lax.axis_index("s")
    wid = s * NUM_CORES + c
    base = wid * EDGES_PER_TILE
    stripe = s * STRIPE

    zeros16 = jnp.zeros((D_EDGE,), jnp.float32)

    def zrow(i, carry):
      zbuf[i, :] = zeros16
      return carry

    lax.fori_loop(0, STRIPE, zrow, 0)
    pltpu.sync_copy(zbuf, acc_sh.at[pl.ds(stripe, STRIPE)])

    @pl.when(s == NUM_SUBCORES - 1)
    def _zero_tail():
      pltpu.sync_copy(
          zbuf.at[pl.ds(0, TAIL)], acc_sh.at[pl.ds(N_NODES - TAIL, TAIL)])

    plsc.subcore_barrier()

    pltpu.sync_copy(recv_hbm.at[pl.ds(base, EDGES_PER_TILE)], idx_all)
    rbufs = (rows0, rows1)
    sems = (sem0, sem1)
    pltpu.async_copy(attr_hbm.at[pl.ds(base, CHUNK)], rows0, sem0)

    def body(jj, carry):
      for b in range(2):
        j = 2 * jj + b
        nxt = (b + 1) % 2

        @pl.when(j + 1 < NFULL)
        def _start_next():
          pltpu.async_copy(
              attr_hbm.at[pl.ds(base + (j + 1) * CHUNK, CHUNK)],
              rbufs[nxt], sems[nxt])

        pltpu.make_async_copy(
            attr_hbm.at[pl.ds(base + j * CHUNK, CHUNK)], rbufs[b],
            sems[b]).wait()
        pltpu.sync_copy(
            rbufs[b], acc_sh.at[idx_all.at[pl.ds(j * CHUNK, CHUNK)]], add=True)
      return carry

    lax.fori_loop(0, NFULL // 2, body, 0)

    offr = base + NFULL * CHUNK
    pltpu.sync_copy(attr_hbm.at[pl.ds(offr, REM)], rowsr_v)
    pltpu.sync_copy(
        rowsr_v, acc_sh.at[idx_all.at[pl.ds(NFULL * CHUNK, REM)]], add=True)

    plsc.subcore_barrier()
    pltpu.sync_copy(
        acc_sh.at[pl.ds(stripe, STRIPE)],
        out_hbm.at[c, pl.ds(stripe, STRIPE)],
    )

    @pl.when(s == NUM_SUBCORES - 1)
    def _write_tail():
      pltpu.sync_copy(
          acc_sh.at[pl.ds(N_NODES - TAIL, TAIL)],
          out_hbm.at[c, pl.ds(N_NODES - TAIL, TAIL)],
      )

  return seg


_sc_segment_sum = _make_sc_segment_sum()

ROW_BLOCK = 1000
N_BLOCKS = N_NODES // ROW_BLOCK


def _mlp_body(x_ref, p_ref, g_ref, w1_ref, b1_ref, w2_ref, b2_ref, o_ref):
  xw = jnp.dot(x_ref[...], w1_ref[0:D_NODE, :],
               preferred_element_type=jnp.float32)
  p = p_ref[0] + p_ref[1]
  pw = jnp.dot(p, w1_ref[D_NODE:D_NODE + D_EDGE, :],
               preferred_element_type=jnp.float32)
  gw = jnp.dot(g_ref[...], w1_ref[D_NODE + D_EDGE:, :],
               preferred_element_type=jnp.float32)
  h = jnp.maximum(xw + pw + gw + b1_ref[...], 0.0)
  o_ref[...] = jnp.dot(h, w2_ref[...],
                       preferred_element_type=jnp.float32) + b2_ref[...]


def _tc_mlp(x, parts, global_attr, W1, b1, W2, b2):
  return pl.pallas_call(
      _mlp_body,
      grid=(N_BLOCKS,),
      in_specs=[
          pl.BlockSpec((ROW_BLOCK, D_NODE), lambda i: (i, 0)),
          pl.BlockSpec((NUM_CORES, ROW_BLOCK, D_EDGE), lambda i: (0, i, 0)),
          pl.BlockSpec((1, D_NODE), lambda i: (0, 0)),
          pl.BlockSpec((D_NODE + D_EDGE + D_NODE, LATENT), lambda i: (0, 0)),
          pl.BlockSpec((1, LATENT), lambda i: (0, 0)),
          pl.BlockSpec((LATENT, D_OUT), lambda i: (0, 0)),
          pl.BlockSpec((1, D_OUT), lambda i: (0, 0)),
      ],
      out_specs=pl.BlockSpec((ROW_BLOCK, D_OUT), lambda i: (i, 0)),
      out_shape=jax.ShapeDtypeStruct((N_NODES, D_OUT), jnp.float32),
  )(x, parts, global_attr, W1, b1, W2, b2)


def kernel(x, edge_index, edge_attr, global_attr, W1, b1, W2, b2):
  recv = edge_index[1].astype(jnp.int32)
  parts = _sc_segment_sum(recv, edge_attr)
  return _tc_mlp(
      x,
      parts,
      global_attr,
      W1,
      b1.reshape(1, LATENT),
      W2,
      b2.reshape(1, D_OUT),
  )


# speedup vs baseline: 5.4042x; 1.0921x over previous
import functools

import jax
import jax.numpy as jnp
from jax import lax
from jax.experimental import pallas as pl
from jax.experimental.pallas import tpu as pltpu
from jax.experimental.pallas import tpu_sc as plsc

N_NODES = 10000
N_EDGES = 320000
D_EDGE = 16
D_NODE = 128
LATENT = 32
D_OUT = 128

NUM_CORES = 2
NUM_SUBCORES = 16
NUM_TILES = NUM_CORES * NUM_SUBCORES

EDGES_PER_TILE = N_EDGES // NUM_TILES
CHUNK = 128
NFULL = EDGES_PER_TILE // CHUNK
REM = EDGES_PER_TILE - NFULL * CHUNK
RING = 6
LOOKAHEAD = 3
STRIPE = 624
TAIL = N_NODES - NUM_SUBCORES * STRIPE


def _make_sc_segment_sum():
  mesh = plsc.VectorSubcoreMesh(core_axis_name="c", subcore_axis_name="s")

  @functools.partial(
      pl.kernel,
      mesh=mesh,
      compiler_params=pltpu.CompilerParams(use_tc_tiling_on_sc=False),
      out_type=jax.ShapeDtypeStruct((NUM_CORES, N_NODES, D_EDGE), jnp.float32),
      scratch_types=(
          [pltpu.VMEM((EDGES_PER_TILE,), jnp.int32)]
          + [pltpu.VMEM((CHUNK, D_EDGE), jnp.float32) for _ in range(RING)]
          + [
              pltpu.VMEM((REM, D_EDGE), jnp.float32),
              pltpu.VMEM((STRIPE, D_EDGE), jnp.float32),
              pltpu.VMEM_SHARED((N_NODES, D_EDGE), jnp.float32),
          ]
          + [pltpu.SemaphoreType.DMA for _ in range(2 * RING)]
      ),
  )
  def seg(ei_hbm, attr_hbm, out_hbm, idx_all, *rest):
    rbufs = rest[0:RING]
    rowsr_v = rest[RING]
    zbuf = rest[RING + 1]
    acc_sh = rest[RING + 2]
    sem_ld = rest[RING + 3:RING + 3 + RING]
    sem_sc = rest[RING + 3 + RING:RING + 3 + 2 * RING]
    c = lax.axis_index("c")
    s = lax.axis_index("s")
    wid = s * NUM_CORES + c
    base = wid * EDGES_PER_TILE
    stripe = s * STRIPE

    zeros16 = jnp.zeros((D_EDGE,), jnp.float32)

    def zrow(i, carry):
      zbuf[i, :] = zeros16
      return carry

    lax.fori_loop(0, STRIPE, zrow, 0)
    pltpu.sync_copy(zbuf, acc_sh.at[pl.ds(stripe, STRIPE)])

    @pl.when(s == NUM_SUBCORES - 1)
    def _zero_tail():
      pltpu.sync_copy(
          zbuf.at[pl.ds(0, TAIL)], acc_sh.at[pl.ds(N_NODES - TAIL, TAIL)])

    plsc.subcore_barrier()

    pltpu.sync_copy(ei_hbm.at[1, pl.ds(base, EDGES_PER_TILE)], idx_all)

    def _load(j, b):
      return pltpu.make_async_copy(
          attr_hbm.at[pl.ds(base + j * CHUNK, CHUNK)], rbufs[b], sem_ld[b])

    def _scat(j, b):
      return pltpu.make_async_copy(
          rbufs[b], acc_sh.at[idx_all.at[pl.ds(j * CHUNK, CHUNK)]], sem_sc[b])

    for k in range(LOOKAHEAD):
      pltpu.async_copy(
          attr_hbm.at[pl.ds(base + k * CHUNK, CHUNK)], rbufs[k], sem_ld[k])

    def body(jj, carry):
      for p in range(RING):
        j = RING * jj + p
        bn = (p + LOOKAHEAD) % RING

        @pl.when(j + LOOKAHEAD < NFULL)
        def _prefetch():
          @pl.when(j >= RING - LOOKAHEAD)
          def _drain_old():
            _scat(j - (RING - LOOKAHEAD), bn).wait()

          pltpu.async_copy(
              attr_hbm.at[pl.ds(base + (j + LOOKAHEAD) * CHUNK, CHUNK)],
              rbufs[bn], sem_ld[bn])

        _load(j, p).wait()
        pltpu.async_copy(
            rbufs[p], acc_sh.at[idx_all.at[pl.ds(j * CHUNK, CHUNK)]],
            sem_sc[p], add=True)
      return carry

    lax.fori_loop(0, NFULL // RING, body, 0)
    for p in range(RING):
      _scat(NFULL - RING + p, p).wait()

    offr = base + NFULL * CHUNK
    pltpu.sync_copy(attr_hbm.at[pl.ds(offr, REM)], rowsr_v)
    pltpu.sync_copy(
        rowsr_v, acc_sh.at[idx_all.at[pl.ds(NFULL * CHUNK, REM)]], add=True)

    plsc.subcore_barrier()
    pltpu.sync_copy(
        acc_sh.at[pl.ds(stripe, STRIPE)],
        out_hbm.at[c, pl.ds(stripe, STRIPE)],
    )

    @pl.when(s == NUM_SUBCORES - 1)
    def _write_tail():
      pltpu.sync_copy(
          acc_sh.at[pl.ds(N_NODES - TAIL, TAIL)],
          out_hbm.at[c, pl.ds(N_NODES - TAIL, TAIL)],
      )

  return seg


_sc_segment_sum = _make_sc_segment_sum()

ROW_BLOCK = 1000
N_BLOCKS = N_NODES // ROW_BLOCK


def _mlp_body(x_ref, p_ref, g_ref, w1_ref, b1_ref, w2_ref, b2_ref, o_ref):
  xw = jnp.dot(x_ref[...], w1_ref[0:D_NODE, :],
               preferred_element_type=jnp.float32)
  p = p_ref[0] + p_ref[1]
  pw = jnp.dot(p, w1_ref[D_NODE:D_NODE + D_EDGE, :],
               preferred_element_type=jnp.float32)
  gw = jnp.dot(g_ref[...], w1_ref[D_NODE + D_EDGE:, :],
               preferred_element_type=jnp.float32)
  h = jnp.maximum(xw + pw + gw + b1_ref[...], 0.0)
  o_ref[...] = jnp.dot(h, w2_ref[...],
                       preferred_element_type=jnp.float32) + b2_ref[...]


def _tc_mlp(x, parts, global_attr, W1, b1, W2, b2):
  return pl.pallas_call(
      _mlp_body,
      grid=(N_BLOCKS,),
      in_specs=[
          pl.BlockSpec((ROW_BLOCK, D_NODE), lambda i: (i, 0)),
          pl.BlockSpec((NUM_CORES, ROW_BLOCK, D_EDGE), lambda i: (0, i, 0)),
          pl.BlockSpec((1, D_NODE), lambda i: (0, 0)),
          pl.BlockSpec((D_NODE + D_EDGE + D_NODE, LATENT), lambda i: (0, 0)),
          pl.BlockSpec((1, LATENT), lambda i: (0, 0)),
          pl.BlockSpec((LATENT, D_OUT), lambda i: (0, 0)),
          pl.BlockSpec((1, D_OUT), lambda i: (0, 0)),
      ],
      out_specs=pl.BlockSpec((ROW_BLOCK, D_OUT), lambda i: (i, 0)),
      out_shape=jax.ShapeDtypeStruct((N_NODES, D_OUT), jnp.float32),
  )(x, parts, global_attr, W1, b1, W2, b2)


def kernel(x, edge_index, edge_attr, global_attr, W1, b1, W2, b2):
  parts = _sc_segment_sum(edge_index.astype(jnp.int32), edge_attr)
  return _tc_mlp(
      x,
      parts,
      global_attr,
      W1,
      b1.reshape(1, LATENT),
      W2,
      b2.reshape(1, D_OUT),
  )
